# baseline (speedup 1.0000x reference)
import functools

import jax
import jax.numpy as jnp
from jax import lax
from jax.experimental import pallas as pl
from jax.experimental.pallas import tpu as pltpu
from jax.experimental.pallas import tpu_sc as plsc

DIM = 64
PDIM = 128
VOCAB_BLOCKS_FULL = 7812
NC = 2
NS = 16
NW = NC * NS

_IOTA16 = tuple(range(16))


def _transpose_block(tin, tout, j, ncols):
    iota = jnp.arange(16, dtype=jnp.int32)
    c0 = jnp.full((16,), 2 * j, jnp.int32)
    c1 = c0 + 1
    for kb in range(8):
        rows = iota + (16 * kb if kb < 4 else 16 * kb - 64)
        col = c0 if kb < 4 else c1
        val = plsc.load_gather(tin, [rows, col])
        tout[j, pl.ds(16 * kb, 16)] = val
    del ncols


def _make_pack(vocab: int):
    mesh = plsc.VectorSubcoreMesh(core_axis_name="c", subcore_axis_name="s")
    vpack = VOCAB_BLOCKS_FULL * 64 + 64

    @functools.partial(
        pl.kernel,
        mesh=mesh,
        out_type=jax.ShapeDtypeStruct((vpack, PDIM), jnp.float32),
        compiler_params=pltpu.CompilerParams(needs_layout_passes=False),
        scratch_types=[
            pltpu.VMEM((DIM, PDIM), jnp.float32),
            pltpu.VMEM((DIM, PDIM), jnp.float32),
            pltpu.SemaphoreType.DMA,
        ],
    )
    def pack(tmain_hbm, ttail_hbm, tpack_hbm, tin_v, tout_v, sem):
        wid = lax.axis_index("s") * NC + lax.axis_index("c")
        nt = 244 + (wid < 4).astype(jnp.int32)

        def do_block(src_copy, rb):
            src_copy()

            def body_j(j, carry):
                _transpose_block(tin_v, tout_v, j, PDIM)
                return carry

            lax.fori_loop(0, DIM, body_j, 0)
            pltpu.sync_copy(tout_v, tpack_hbm.at[pl.ds(rb, DIM)])

        def body(t, carry):
            b = wid + NW * t
            cb = b * PDIM

            def copy_in():
                pltpu.sync_copy(tmain_hbm.at[:, pl.ds(cb, PDIM)], tin_v)

            do_block(copy_in, b * DIM)
            return carry

        lax.fori_loop(0, nt, body, 0)

        @pl.when(wid == NW - 1)
        def _():
            def copy_in():
                pltpu.sync_copy(ttail_hbm, tin_v)

            do_block(copy_in, VOCAB_BLOCKS_FULL * DIM)

    return pack


def _make_gather(batch: int):
    b_per_w = batch // NW
    n_blocks = b_per_w // PDIM
    vpack = VOCAB_BLOCKS_FULL * 64 + 64

    mesh = plsc.VectorSubcoreMesh(core_axis_name="c", subcore_axis_name="s")

    @functools.partial(
        pl.kernel,
        mesh=mesh,
        out_type=jax.ShapeDtypeStruct((50, DIM, 16384), jnp.float32),
        compiler_params=pltpu.CompilerParams(needs_layout_passes=False),
        scratch_types=[
            pltpu.VMEM((b_per_w,), jnp.int32),
            pltpu.VMEM((b_per_w,), jnp.int32),
            pltpu.VMEM((PDIM, PDIM), jnp.float32),
            pltpu.VMEM((DIM, PDIM), jnp.float32),
            pltpu.SemaphoreType.DMA,
            pltpu.SemaphoreType.DMA,
        ],
    )
    def gather(idx2_hbm, p64_hbm, tpack_hbm, out_hbm, idx_v, p64_v,
               rows_v, tout_v, sg, sw):
        wid = lax.axis_index("s") * NC + lax.axis_index("c")
        base = wid * b_per_w
        pltpu.sync_copy(idx2_hbm.at[pl.ds(base, b_per_w)], idx_v)
        pltpu.sync_copy(p64_hbm.at[pl.ds(base, b_per_w)], p64_v)
        iota = jnp.arange(16, dtype=jnp.int32)

        def body(k, carry):
            blk = wid * n_blocks + k
            h = blk // PDIM
            bb = blk % PDIM
            idx_sl = idx_v.at[pl.ds(k * PDIM, PDIM)]
            pltpu.async_copy(tpack_hbm.at[idx_sl], rows_v, sg).wait()

            pcols = [p64_v[pl.ds(k * PDIM + 16 * jb, 16)] for jb in range(8)]

            def body_d(d, carry2):
                for jb in range(8):
                    rows = iota + 16 * jb
                    val = plsc.load_gather(rows_v, [rows, pcols[jb] + d])
                    tout_v[d, pl.ds(16 * jb, 16)] = val
                return carry2

            lax.fori_loop(0, DIM, body_d, 0)
            pltpu.sync_copy(tout_v, out_hbm.at[h, :, pl.ds(bb * PDIM, PDIM)])
            return carry

        lax.fori_loop(0, n_blocks, body, 0)

    return gather


def kernel(tokens, table):
    b, h = tokens.shape
    vocab = table.shape[0]
    flat = tokens.astype(jnp.int32).T.reshape(-1)
    idx2 = flat >> 1
    p64 = (flat & 1) << 6
    table_t = table.T
    ttail = jnp.pad(table_t[:, VOCAB_BLOCKS_FULL * PDIM :],
                    ((0, 0), (0, PDIM - (vocab - VOCAB_BLOCKS_FULL * PDIM))))
    tpack = _make_pack(vocab)(table_t, ttail)
    out3 = _make_gather(b * h)(idx2, p64, tpack)
    return jnp.transpose(out3, (2, 0, 1))


# speedup vs baseline: 1.6168x; 1.6168x over previous
import functools

import jax
import jax.numpy as jnp
from jax import lax
from jax.experimental import pallas as pl
from jax.experimental.pallas import tpu as pltpu
from jax.experimental.pallas import tpu_sc as plsc

DIM = 64
PDIM = 128
HALF = 500000
PACK_C = 400
PACK_NBLK = HALF // PACK_C
NC = 2
NS = 16
NW = NC * NS


def _make_pack():
    mesh = plsc.VectorSubcoreMesh(core_axis_name="c", subcore_axis_name="s")

    @functools.partial(
        pl.kernel,
        mesh=mesh,
        out_type=jax.ShapeDtypeStruct((HALF, PDIM), jnp.float32),
        compiler_params=pltpu.CompilerParams(needs_layout_passes=False),
        scratch_types=[
            pltpu.VMEM((PACK_C, PDIM), jnp.float32),
            pltpu.SemaphoreType.DMA,
        ],
    )
    def pack(table_hbm, tpack_hbm, buf_v, sem):
        wid = lax.axis_index("s") * NC + lax.axis_index("c")
        base_nt = PACK_NBLK // NW
        nt = base_nt + (wid < PACK_NBLK - base_nt * NW).astype(jnp.int32)

        def body(t, carry):
            b = wid + NW * t
            r0 = b * PACK_C
            pltpu.sync_copy(table_hbm.at[pl.ds(r0, PACK_C), :],
                            buf_v.at[:, pl.ds(0, DIM)])
            pltpu.sync_copy(table_hbm.at[pl.ds(r0 + HALF, PACK_C), :],
                            buf_v.at[:, pl.ds(DIM, DIM)])
            pltpu.sync_copy(buf_v, tpack_hbm.at[pl.ds(r0, PACK_C)])
            return carry

        lax.fori_loop(0, nt, body, 0)

    return pack


def _make_gather(batch: int):
    b_per_w = batch // NW
    n_blocks = b_per_w // PDIM

    mesh = plsc.VectorSubcoreMesh(core_axis_name="c", subcore_axis_name="s")

    @functools.partial(
        pl.kernel,
        mesh=mesh,
        out_type=jax.ShapeDtypeStruct((50, DIM, 16384), jnp.float32),
        compiler_params=pltpu.CompilerParams(needs_layout_passes=False),
        scratch_types=[
            pltpu.VMEM((b_per_w,), jnp.int32),
            pltpu.VMEM((PDIM, PDIM), jnp.float32),
            pltpu.VMEM((DIM, PDIM), jnp.float32),
            pltpu.SemaphoreType.DMA,
            pltpu.SemaphoreType.DMA,
        ],
    )
    def gather(idx2_hbm, tpack_hbm, out_hbm, idx_v,
               rows_v, tout_v, sg, sw):
        wid = lax.axis_index("s") * NC + lax.axis_index("c")
        base = wid * b_per_w
        pltpu.sync_copy(idx2_hbm.at[pl.ds(base, b_per_w)], idx_v)
        iota = jnp.arange(16, dtype=jnp.int32)

        def body(k, carry):
            blk = wid * n_blocks + k
            h = blk // PDIM
            bb = blk % PDIM
            idx_sl = idx_v.at[pl.ds(k * PDIM, PDIM)]
            pltpu.async_copy(tpack_hbm.at[idx_sl], rows_v, sg).wait()

            rowvs = [iota + 16 * jb for jb in range(8)]
            for d in range(DIM):
                dcol = jnp.full((16,), d, jnp.int32)
                for jb in range(8):
                    val = plsc.load_gather(rows_v, [rowvs[jb], dcol])
                    tout_v[d, pl.ds(16 * jb, 16)] = val
            pltpu.sync_copy(tout_v, out_hbm.at[h, :, pl.ds(bb * PDIM, PDIM)])
            return carry

        lax.fori_loop(0, n_blocks, body, 0)

    return gather


def kernel(tokens, table):
    b, h = tokens.shape
    flat = tokens.astype(jnp.int32).T.reshape(-1)
    tpack = jnp.pad(table, ((0, 0), (0, PDIM - DIM)))
    out3 = _make_gather(b * h)(flat, tpack)
    return jnp.transpose(out3, (2, 0, 1))


# speedup vs baseline: 2.2221x; 1.3744x over previous
import functools

import jax
import jax.numpy as jnp
from jax import lax
from jax.experimental import pallas as pl
from jax.experimental.pallas import tpu as pltpu
from jax.experimental.pallas import tpu_sc as plsc

DIM = 64
PDIM = 128
HALF = 500000
PACK_C = 400
PACK_NBLK = HALF // PACK_C
NC = 2
NS = 16
NW = NC * NS


def _make_pack():
    mesh = plsc.VectorSubcoreMesh(core_axis_name="c", subcore_axis_name="s")

    @functools.partial(
        pl.kernel,
        mesh=mesh,
        out_type=jax.ShapeDtypeStruct((HALF, PDIM), jnp.float32),
        compiler_params=pltpu.CompilerParams(needs_layout_passes=False),
        scratch_types=[
            pltpu.VMEM((PACK_C, PDIM), jnp.float32),
            pltpu.SemaphoreType.DMA,
        ],
    )
    def pack(table_hbm, tpack_hbm, buf_v, sem):
        wid = lax.axis_index("s") * NC + lax.axis_index("c")
        base_nt = PACK_NBLK // NW
        nt = base_nt + (wid < PACK_NBLK - base_nt * NW).astype(jnp.int32)

        def body(t, carry):
            b = wid + NW * t
            r0 = b * PACK_C
            pltpu.sync_copy(table_hbm.at[pl.ds(r0, PACK_C), :],
                            buf_v.at[:, pl.ds(0, DIM)])
            pltpu.sync_copy(table_hbm.at[pl.ds(r0 + HALF, PACK_C), :],
                            buf_v.at[:, pl.ds(DIM, DIM)])
            pltpu.sync_copy(buf_v, tpack_hbm.at[pl.ds(r0, PACK_C)])
            return carry

        lax.fori_loop(0, nt, body, 0)

    return pack


def _make_gather(batch: int):
    b_per_w = batch // NW
    n_blocks = b_per_w // PDIM

    mesh = plsc.VectorSubcoreMesh(core_axis_name="c", subcore_axis_name="s")

    @functools.partial(
        pl.kernel,
        mesh=mesh,
        out_type=jax.ShapeDtypeStruct((50, DIM, 16384), jnp.float32),
        compiler_params=pltpu.CompilerParams(needs_layout_passes=False),
        scratch_types=[
            pltpu.VMEM((b_per_w,), jnp.int32),
            pltpu.VMEM((PDIM, PDIM), jnp.float32),
            pltpu.VMEM((DIM, PDIM), jnp.float32),
            pltpu.SemaphoreType.DMA,
            pltpu.SemaphoreType.DMA,
        ],
    )
    def gather(idx2_hbm, tpack_hbm, out_hbm, idx_v,
               rows_v, tout_v, sg, sw):
        wid = lax.axis_index("s") * NC + lax.axis_index("c")
        base = wid * b_per_w
        pltpu.sync_copy(idx2_hbm.at[pl.ds(base, b_per_w)], idx_v)
        iota = jnp.arange(16, dtype=jnp.int32)

        def body(k, carry):
            blk = wid * n_blocks + k
            h = blk // PDIM
            bb = blk % PDIM
            idx_sl = idx_v.at[pl.ds(k * PDIM, PDIM)]
            pltpu.async_copy(tpack_hbm.at[idx_sl], rows_v, sg).wait()

            rowvs = [iota + 16 * jb for jb in range(8)]

            @plsc.parallel_loop(0, DIM, 1, unroll=8)
            def _(d):
                dcol = jnp.full((16,), d, jnp.int32)
                for jb in range(8):
                    val = plsc.load_gather(rows_v, [rowvs[jb], dcol])
                    tout_v[d, pl.ds(16 * jb, 16)] = val
            pltpu.sync_copy(tout_v, out_hbm.at[h, :, pl.ds(bb * PDIM, PDIM)])
            return carry

        lax.fori_loop(0, n_blocks, body, 0)

    return gather


def kernel(tokens, table):
    b, h = tokens.shape
    flat = tokens.astype(jnp.int32).T.reshape(-1)
    tpack = jnp.pad(table, ((0, 0), (0, PDIM - DIM)))
    out3 = _make_gather(b * h)(flat, tpack)
    return jnp.transpose(out3, (2, 0, 1))


# speedup vs baseline: 2.8923x; 1.3016x over previous
import functools

import jax
import jax.numpy as jnp
from jax import lax
from jax.experimental import pallas as pl
from jax.experimental.pallas import tpu as pltpu
from jax.experimental.pallas import tpu_sc as plsc

DIM = 64
PDIM = 128
HALF = 500000
PACK_C = 400
PACK_NBLK = HALF // PACK_C
NC = 2
NS = 16
NW = NC * NS


def _make_pack():
    mesh = plsc.VectorSubcoreMesh(core_axis_name="c", subcore_axis_name="s")

    @functools.partial(
        pl.kernel,
        mesh=mesh,
        out_type=jax.ShapeDtypeStruct((HALF, PDIM), jnp.float32),
        compiler_params=pltpu.CompilerParams(needs_layout_passes=False),
        scratch_types=[
            pltpu.VMEM((PACK_C, PDIM), jnp.float32),
            pltpu.SemaphoreType.DMA,
        ],
    )
    def pack(table_hbm, tpack_hbm, buf_v, sem):
        wid = lax.axis_index("s") * NC + lax.axis_index("c")
        base_nt = PACK_NBLK // NW
        nt = base_nt + (wid < PACK_NBLK - base_nt * NW).astype(jnp.int32)

        def body(t, carry):
            b = wid + NW * t
            r0 = b * PACK_C
            pltpu.sync_copy(table_hbm.at[pl.ds(r0, PACK_C), :],
                            buf_v.at[:, pl.ds(0, DIM)])
            pltpu.sync_copy(table_hbm.at[pl.ds(r0 + HALF, PACK_C), :],
                            buf_v.at[:, pl.ds(DIM, DIM)])
            pltpu.sync_copy(buf_v, tpack_hbm.at[pl.ds(r0, PACK_C)])
            return carry

        lax.fori_loop(0, nt, body, 0)

    return pack


def _make_gather(batch: int):
    b_per_w = batch // NW
    n_blocks = b_per_w // PDIM

    mesh = plsc.VectorSubcoreMesh(core_axis_name="c", subcore_axis_name="s")

    @functools.partial(
        pl.kernel,
        mesh=mesh,
        out_type=jax.ShapeDtypeStruct((50, DIM, 16384), jnp.float32),
        compiler_params=pltpu.CompilerParams(needs_layout_passes=False),
        scratch_types=[
            pltpu.VMEM((b_per_w,), jnp.int32),
            pltpu.VMEM((PDIM, PDIM), jnp.float32),
            pltpu.VMEM((PDIM, PDIM), jnp.float32),
            pltpu.VMEM((DIM, PDIM), jnp.float32),
            pltpu.VMEM((DIM, PDIM), jnp.float32),
            pltpu.SemaphoreType.DMA,
            pltpu.SemaphoreType.DMA,
            pltpu.SemaphoreType.DMA,
            pltpu.SemaphoreType.DMA,
        ],
    )
    def gather(idx2_hbm, tpack_hbm, out_hbm, idx_v,
               rows0, rows1, tout0, tout1, sg0, sg1, sw0, sw1):
        wid = lax.axis_index("s") * NC + lax.axis_index("c")
        base = wid * b_per_w
        pltpu.sync_copy(idx2_hbm.at[pl.ds(base, b_per_w)], idx_v)
        iota = jnp.arange(16, dtype=jnp.int32)
        rowvs = [iota + 16 * jb for jb in range(8)]

        def g_copy(k, rows, sg):
            idx_sl = idx_v.at[pl.ds(k * PDIM, PDIM)]
            return pltpu.make_async_copy(tpack_hbm.at[idx_sl], rows, sg)

        def out_copy(k, tout, sw):
            blk = wid * n_blocks + k
            h = blk // PDIM
            bb = blk % PDIM
            return pltpu.make_async_copy(
                tout, out_hbm.at[h, :, pl.ds(bb * PDIM, PDIM)], sw)

        def transpose(rows_v, tout_v):
            @plsc.parallel_loop(0, DIM, 1, unroll=8)
            def _(d):
                dcol = jnp.full((16,), d, jnp.int32)
                for jb in range(8):
                    val = plsc.load_gather(rows_v, [rowvs[jb], dcol])
                    tout_v[d, pl.ds(16 * jb, 16)] = val

        def stage(k, g, rows_a, sg_a, rows_b, sg_b, tout_a, sw_a, last):
            @pl.when(jnp.logical_not(last))
            def _():
                g_copy(k + 1, rows_b, sg_b).start()
            g_copy(k, rows_a, sg_a).wait()

            @pl.when(g > 0)
            def _():
                out_copy(k - 2, tout_a, sw_a).wait()
            transpose(rows_a, tout_a)
            out_copy(k, tout_a, sw_a).start()

        def body(g, carry):
            k0 = 2 * g
            f = jnp.bool_(False)
            stage(k0, g, rows0, sg0, rows1, sg1, tout0, sw0, f)
            stage(k0 + 1, g, rows1, sg1, rows0, sg0, tout1, sw1,
                  g == n_blocks // 2 - 1)
            return carry

        g_copy(0, rows0, sg0).start()
        lax.fori_loop(0, n_blocks // 2, body, 0)
        out_copy(n_blocks - 2, tout0, sw0).wait()
        out_copy(n_blocks - 1, tout1, sw1).wait()

    return gather


def kernel(tokens, table):
    b, h = tokens.shape
    flat = tokens.astype(jnp.int32).T.reshape(-1)
    tpack = jnp.pad(table, ((0, 0), (0, PDIM - DIM)))
    out3 = _make_gather(b * h)(flat, tpack)
    return jnp.transpose(out3, (2, 0, 1))
